# baseline (speedup 1.0000x reference)
import jax
import jax.numpy as jnp
from jax import lax
from jax.experimental import pallas as pl
from jax.experimental.pallas import tpu as pltpu
from jax.experimental.pallas import tpu_sc as plsc

_NC = 2
_NS = 16
_NW = _NC * _NS

_NBUF = 4
_LAG = 2
_UNROLL = 16


def _make_sc_gather(bsz: int, seq: int, dim: int):
    assert bsz % _NW == 0 and seq >= _UNROLL
    b_per_w = bsz // _NW
    assert b_per_w % _NBUF == 0 and _LAG < _NBUF

    mesh = plsc.VectorSubcoreMesh(core_axis_name="c", subcore_axis_name="s")

    scratch = [
        pltpu.VMEM((_NBUF, seq, dim), jnp.float32),
        pltpu.VMEM((_NBUF, seq), jnp.int32),
    ] + [pltpu.SemaphoreType.DMA] * (2 * _NBUF)

    def body(idx_hbm, table_hbm, out_hbm, rows_v, idx_v, *sems):
        gsem = sems[:_NBUF]
        ssem = sems[_NBUF:]
        wid = lax.axis_index("s") * _NC + lax.axis_index("c")
        b_base = wid * b_per_w

        def gather_start(c, s):
            pltpu.sync_copy(idx_hbm.at[b_base + c], idx_v.at[s])

            ngroups = seq // _UNROLL
            tail = seq - ngroups * _UNROLL

            def issue(g, _):
                toks = idx_v[s, pl.ds(g * _UNROLL, _UNROLL)]
                for u in range(_UNROLL):
                    pltpu.async_copy(
                        table_hbm.at[toks[u]],
                        rows_v.at[s, g * _UNROLL + u], gsem[s])
                return 0

            lax.fori_loop(0, ngroups, issue, 0)
            if tail:
                toks = idx_v[s, pl.ds(seq - _UNROLL, _UNROLL)]
                for u in range(_UNROLL - tail, _UNROLL):
                    pltpu.async_copy(
                        table_hbm.at[toks[u]],
                        rows_v.at[s, seq - _UNROLL + u], gsem[s])

        def gather_wait(s):
            pltpu.make_async_copy(
                table_hbm.at[pl.ds(0, seq)], rows_v.at[s], gsem[s]).wait()

        def scatter_start(c, s):
            pltpu.async_copy(rows_v.at[s], out_hbm.at[b_base + c], ssem[s])

        def scatter_wait(c, s):
            pltpu.make_async_copy(
                rows_v.at[s], out_hbm.at[b_base + c], ssem[s]).wait()

        for c in range(_NBUF):
            gather_start(c, c)
            if c >= _LAG:
                cc = c - _LAG
                gather_wait(cc)
                scatter_start(cc, cc)

        def outer_body(g, _):
            for s in range(_NBUF):
                c = g * _NBUF + s
                scatter_wait(c - _NBUF, s)
                gather_start(c, s)
                cc = c - _LAG
                sc = (s - _LAG) % _NBUF
                gather_wait(sc)
                scatter_start(cc, sc)
            return 0

        lax.fori_loop(1, b_per_w // _NBUF, outer_body, 0)

        for cc in range(b_per_w - _LAG, b_per_w):
            gather_wait(cc % _NBUF)
            scatter_start(cc, cc % _NBUF)
        for c in range(b_per_w - _NBUF, b_per_w):
            scatter_wait(c, c % _NBUF)

    return pl.kernel(
        body,
        out_type=jax.ShapeDtypeStruct((bsz, seq, dim), jnp.float32),
        mesh=mesh,
        scratch_types=scratch,
        compiler_params=pltpu.CompilerParams(use_tc_tiling_on_sc=True),
    )


_NSPLIT = 2


def _split_parts(seq: int, n: int):
    assert seq % 8 == 0
    blocks = seq // 8
    parts = [8 * (blocks // n + (i < blocks % n)) for i in range(n)]
    return [p for p in parts if p]


@jax.jit
def kernel(token_ids, weight):
    bsz, seq = token_ids.shape
    num, dim = weight.shape
    idx = token_ids.astype(jnp.int32)
    outs = []
    t0 = 0
    for part in _split_parts(seq, _NSPLIT):
        outs.append(
            _make_sc_gather(bsz, part, dim)(idx[:, t0:t0 + part], weight))
        t0 += part
    if len(outs) == 1:
        return outs[0]
    return jnp.concatenate(outs, axis=1)


# speedup vs baseline: 1.1360x; 1.1360x over previous
import jax
import jax.numpy as jnp
from jax import lax
from jax.experimental import pallas as pl
from jax.experimental.pallas import tpu as pltpu
from jax.experimental.pallas import tpu_sc as plsc

_NC = 2
_NS = 16
_NW = _NC * _NS

_NBUF = 4
_LAG = 3
_UNROLL = 16


def _make_sc_gather(bsz: int, seq: int, dim: int):
    assert bsz % _NW == 0 and seq >= _UNROLL
    b_per_w = bsz // _NW
    assert b_per_w % _NBUF == 0 and _LAG < _NBUF

    mesh = plsc.VectorSubcoreMesh(core_axis_name="c", subcore_axis_name="s")

    scratch = [
        pltpu.VMEM((_NBUF, seq, dim), jnp.float32),
        pltpu.VMEM((_NBUF, seq), jnp.int32),
    ] + [pltpu.SemaphoreType.DMA] * (2 * _NBUF)

    def body(idx_hbm, table_hbm, out_hbm, rows_v, idx_v, *sems):
        gsem = sems[:_NBUF]
        ssem = sems[_NBUF:]
        wid = lax.axis_index("s") * _NC + lax.axis_index("c")
        b_base = wid * b_per_w

        def gather_start(c, s):
            pltpu.sync_copy(idx_hbm.at[b_base + c], idx_v.at[s])

            ngroups = seq // _UNROLL
            tail = seq - ngroups * _UNROLL

            def issue(g, _):
                toks = idx_v[s, pl.ds(g * _UNROLL, _UNROLL)]
                for u in range(_UNROLL):
                    pltpu.async_copy(
                        table_hbm.at[toks[u]],
                        rows_v.at[s, g * _UNROLL + u], gsem[s])
                return 0

            lax.fori_loop(0, ngroups, issue, 0)
            if tail:
                toks = idx_v[s, pl.ds(seq - _UNROLL, _UNROLL)]
                for u in range(_UNROLL - tail, _UNROLL):
                    pltpu.async_copy(
                        table_hbm.at[toks[u]],
                        rows_v.at[s, seq - _UNROLL + u], gsem[s])

        def gather_wait(s):
            pltpu.make_async_copy(
                table_hbm.at[pl.ds(0, seq)], rows_v.at[s], gsem[s]).wait()

        def scatter_start(c, s):
            pltpu.async_copy(rows_v.at[s], out_hbm.at[b_base + c], ssem[s])

        def scatter_wait(c, s):
            pltpu.make_async_copy(
                rows_v.at[s], out_hbm.at[b_base + c], ssem[s]).wait()

        for c in range(_NBUF):
            gather_start(c, c)
            if c >= _LAG:
                cc = c - _LAG
                gather_wait(cc)
                scatter_start(cc, cc)

        def outer_body(g, _):
            for s in range(_NBUF):
                c = g * _NBUF + s
                scatter_wait(c - _NBUF, s)
                gather_start(c, s)
                cc = c - _LAG
                sc = (s - _LAG) % _NBUF
                gather_wait(sc)
                scatter_start(cc, sc)
            return 0

        lax.fori_loop(1, b_per_w // _NBUF, outer_body, 0)

        for cc in range(b_per_w - _LAG, b_per_w):
            gather_wait(cc % _NBUF)
            scatter_start(cc, cc % _NBUF)
        for c in range(b_per_w - _NBUF, b_per_w):
            scatter_wait(c, c % _NBUF)

    return pl.kernel(
        body,
        out_type=jax.ShapeDtypeStruct((bsz, seq, dim), jnp.float32),
        mesh=mesh,
        scratch_types=scratch,
        compiler_params=pltpu.CompilerParams(use_tc_tiling_on_sc=True),
    )


_NSPLIT = 1


def _split_parts(seq: int, n: int):
    assert seq % 8 == 0
    blocks = seq // 8
    parts = [8 * (blocks // n + (i < blocks % n)) for i in range(n)]
    return [p for p in parts if p]


@jax.jit
def kernel(token_ids, weight):
    bsz, seq = token_ids.shape
    num, dim = weight.shape
    idx = token_ids.astype(jnp.int32)
    outs = []
    t0 = 0
    for part in _split_parts(seq, _NSPLIT):
        outs.append(
            _make_sc_gather(bsz, part, dim)(idx[:, t0:t0 + part], weight))
        t0 += part
    if len(outs) == 1:
        return outs[0]
    return jnp.concatenate(outs, axis=1)


# speedup vs baseline: 1.1785x; 1.0374x over previous
import jax
import jax.numpy as jnp
from jax import lax
from jax.experimental import pallas as pl
from jax.experimental.pallas import tpu as pltpu
from jax.experimental.pallas import tpu_sc as plsc

_NC = 2
_NS = 16
_NW = _NC * _NS

_NBUF = 4
_LAG = 2
_UNROLL = 16


def _make_sc_gather(bsz: int, seq: int, dim: int):
    assert bsz % _NW == 0 and seq >= _UNROLL and seq % 8 == 0
    b_per_w = bsz // _NW
    assert b_per_w % _NBUF == 0 and _LAG < _NBUF
    n_ids = b_per_w * seq

    mesh = plsc.VectorSubcoreMesh(core_axis_name="c", subcore_axis_name="s")

    scratch = [
        pltpu.VMEM((_NBUF, seq, dim), jnp.float32),
        pltpu.VMEM((n_ids,), jnp.int32),
    ] + [pltpu.SemaphoreType.DMA] * (2 * _NBUF)

    def body(idx_hbm, table_hbm, out_hbm, rows_v, idx_v, *sems):
        gsem = sems[:_NBUF]
        ssem = sems[_NBUF:]
        wid = lax.axis_index("s") * _NC + lax.axis_index("c")
        b_base = wid * b_per_w

        pltpu.sync_copy(idx_hbm.at[pl.ds(wid * n_ids, n_ids)], idx_v)

        ngroups = seq // _UNROLL
        tail = seq - ngroups * _UNROLL

        def gather_start(c, s):
            base = c * seq

            def issue(g, _):
                toks = idx_v[pl.ds(base + g * _UNROLL, _UNROLL)]
                for u in range(_UNROLL):
                    pltpu.async_copy(
                        table_hbm.at[toks[u]],
                        rows_v.at[s, g * _UNROLL + u], gsem[s])
                return 0

            lax.fori_loop(0, ngroups, issue, 0)
            if tail:
                toks = idx_v[pl.ds(base + seq - _UNROLL, _UNROLL)]
                for u in range(_UNROLL - tail, _UNROLL):
                    pltpu.async_copy(
                        table_hbm.at[toks[u]],
                        rows_v.at[s, seq - _UNROLL + u], gsem[s])

        def gather_wait(s):
            pltpu.make_async_copy(
                table_hbm.at[pl.ds(0, seq)], rows_v.at[s], gsem[s]).wait()

        def scatter_start(c, s):
            pltpu.async_copy(rows_v.at[s], out_hbm.at[b_base + c], ssem[s])

        def scatter_wait(c, s):
            pltpu.make_async_copy(
                rows_v.at[s], out_hbm.at[b_base + c], ssem[s]).wait()

        for c in range(_NBUF):
            gather_start(c, c)
            if c >= _LAG:
                cc = c - _LAG
                gather_wait(cc)
                scatter_start(cc, cc)

        def outer_body(g, _):
            for s in range(_NBUF):
                c = g * _NBUF + s
                scatter_wait(c - _NBUF, s)
                gather_start(c, s)
                cc = c - _LAG
                sc = (s - _LAG) % _NBUF
                gather_wait(sc)
                scatter_start(cc, sc)
            return 0

        lax.fori_loop(1, b_per_w // _NBUF, outer_body, 0)

        for cc in range(b_per_w - _LAG, b_per_w):
            gather_wait(cc % _NBUF)
            scatter_start(cc, cc % _NBUF)
        for c in range(b_per_w - _NBUF, b_per_w):
            scatter_wait(c, c % _NBUF)

    return pl.kernel(
        body,
        out_type=jax.ShapeDtypeStruct((bsz, seq, dim), jnp.float32),
        mesh=mesh,
        scratch_types=scratch,
        compiler_params=pltpu.CompilerParams(use_tc_tiling_on_sc=True),
    )


@jax.jit
def kernel(token_ids, weight):
    bsz, seq = token_ids.shape
    num, dim = weight.shape
    idx = token_ids.astype(jnp.int32).reshape(-1)
    return _make_sc_gather(bsz, seq, dim)(idx, weight)


# speedup vs baseline: 1.5764x; 1.3376x over previous
import jax
import jax.numpy as jnp
from jax import lax
from jax.experimental import pallas as pl
from jax.experimental.pallas import tpu as pltpu
from jax.experimental.pallas import tpu_sc as plsc

_NC = 2
_NS = 16
_NW = _NC * _NS

_NBUF = 4
_LAG = 2
_UNROLL = 16


def _make_sc_gather(bsz: int, seq: int, dim: int):
    assert bsz % _NW == 0 and seq >= _UNROLL and seq % 8 == 0
    b_per_w = bsz // _NW
    assert b_per_w % _NBUF == 0 and _LAG < _NBUF
    n_ids = b_per_w * seq

    mesh = plsc.VectorSubcoreMesh(core_axis_name="c", subcore_axis_name="s")

    scratch = [
        pltpu.VMEM((_NBUF, seq // 2, 2 * dim), jnp.float32),
        pltpu.VMEM((n_ids,), jnp.int32),
    ] + [pltpu.SemaphoreType.DMA] * (2 * _NBUF)

    def body(idx_hbm, table_hbm, out_hbm, rows_v, idx_v, *sems):
        gsem = sems[:_NBUF]
        ssem = sems[_NBUF:]
        wid = lax.axis_index("s") * _NC + lax.axis_index("c")
        b_base = wid * b_per_w

        pltpu.sync_copy(idx_hbm.at[pl.ds(wid * n_ids, n_ids)], idx_v)

        ngroups = seq // _UNROLL
        tail = seq - ngroups * _UNROLL

        def gather_start(c, s):
            base = c * seq

            def issue(g, _):
                toks = idx_v[pl.ds(base + g * _UNROLL, _UNROLL)]
                for u in range(_UNROLL):
                    i2 = g * (_UNROLL // 2) + u // 2
                    pltpu.async_copy(
                        table_hbm.at[toks[u]],
                        rows_v.at[s, i2, pl.ds((u % 2) * dim, dim)], gsem[s])
                return 0

            lax.fori_loop(0, ngroups, issue, 0)
            if tail:
                toks = idx_v[pl.ds(base + seq - _UNROLL, _UNROLL)]
                for u in range(_UNROLL - tail, _UNROLL):
                    i2 = (seq - _UNROLL + u) // 2
                    pltpu.async_copy(
                        table_hbm.at[toks[u]],
                        rows_v.at[s, i2, pl.ds((u % 2) * dim, dim)], gsem[s])

        def gather_wait(s):
            pltpu.make_async_copy(
                out_hbm.at[b_base], rows_v.at[s], gsem[s]).wait()

        def scatter_start(c, s):
            pltpu.async_copy(rows_v.at[s], out_hbm.at[b_base + c], ssem[s])

        def scatter_wait(c, s):
            pltpu.make_async_copy(
                rows_v.at[s], out_hbm.at[b_base + c], ssem[s]).wait()

        for c in range(_NBUF):
            gather_start(c, c)
            if c >= _LAG:
                cc = c - _LAG
                gather_wait(cc)
                scatter_start(cc, cc)

        def outer_body(g, _):
            for s in range(_NBUF):
                c = g * _NBUF + s
                scatter_wait(c - _NBUF, s)
                gather_start(c, s)
                cc = c - _LAG
                sc = (s - _LAG) % _NBUF
                gather_wait(sc)
                scatter_start(cc, sc)
            return 0

        lax.fori_loop(1, b_per_w // _NBUF, outer_body, 0)

        for cc in range(b_per_w - _LAG, b_per_w):
            gather_wait(cc % _NBUF)
            scatter_start(cc, cc % _NBUF)
        for c in range(b_per_w - _NBUF, b_per_w):
            scatter_wait(c, c % _NBUF)

    return pl.kernel(
        body,
        out_type=jax.ShapeDtypeStruct((bsz, seq // 2, 2 * dim), jnp.float32),
        mesh=mesh,
        scratch_types=scratch,
        compiler_params=pltpu.CompilerParams(use_tc_tiling_on_sc=True),
    )


@jax.jit
def kernel(token_ids, weight):
    bsz, seq = token_ids.shape
    num, dim = weight.shape
    idx = token_ids.astype(jnp.int32).reshape(-1)
    packed = _make_sc_gather(bsz, seq, dim)(idx, weight)
    return packed.reshape(bsz, seq, dim)
